# Initial kernel scaffold; baseline (speedup 1.0000x reference)
import jax
import jax.numpy as jnp
from jax.experimental import pallas as pl

N = 16384
K = 1024
D = 64
BETA = 0.25
BLK = 1024


def _tc_body(z_ref, et_ref, z2_ref, e2_ref, e_ref,
             nearest_ref, zq_ref, loss_ref):
    i = pl.program_id(0)
    zb = z_ref[...]
    m = jax.lax.dot_general(
        zb, et_ref[...], (((1,), (0,)), ((), ())),
        preferred_element_type=jnp.float32)
    t1 = z2_ref[...] + e2_ref[...]
    d2 = t1 - 2.0 * m
    d = jnp.sqrt(jnp.maximum(d2, 0.0))
    nearest = jnp.argmin(d, axis=1)
    nearest_ref[0, ...] = nearest.reshape(1, BLK)
    onehot = (jax.lax.broadcasted_iota(jnp.int32, (BLK, K), 1)
              == nearest[:, None]).astype(jnp.float32)
    zq_ref[...] = jax.lax.dot_general(
        onehot, e_ref[...], (((1,), (0,)), ((), ())),
        preferred_element_type=jnp.float32)
    d2min = jnp.min(jnp.maximum(d2, 0.0), axis=1)

    @pl.when(i == 0)
    def _():
        loss_ref[0, 0] = 0.0

    loss_ref[0, 0] += jnp.sum(d2min)


def kernel(z, embeddings):
    z2 = jnp.sum(z * z, axis=1, keepdims=True)
    e2 = jnp.sum(embeddings * embeddings, axis=1)[None, :]
    et = embeddings.T
    grid = N // BLK
    nearest3, zq, loss_sum = pl.pallas_call(
        _tc_body,
        grid=(grid,),
        in_specs=[
            pl.BlockSpec((BLK, D), lambda i: (i, 0)),
            pl.BlockSpec((D, K), lambda i: (0, 0)),
            pl.BlockSpec((BLK, 1), lambda i: (i, 0)),
            pl.BlockSpec((1, K), lambda i: (0, 0)),
            pl.BlockSpec((K, D), lambda i: (0, 0)),
        ],
        out_specs=[
            pl.BlockSpec((1, 1, BLK), lambda i: (i, 0, 0)),
            pl.BlockSpec((BLK, D), lambda i: (i, 0)),
            pl.BlockSpec((1, 1), lambda i: (0, 0)),
        ],
        out_shape=[
            jax.ShapeDtypeStruct((grid, 1, BLK), jnp.int32),
            jax.ShapeDtypeStruct((N, D), jnp.float32),
            jax.ShapeDtypeStruct((1, 1), jnp.float32),
        ],
    )(z, et, z2, e2, embeddings)
    nearest = nearest3.reshape(N)
    loss = loss_sum[0, 0] * ((1.0 + BETA) / (N * D))
    z_q_ste = z + jax.lax.stop_gradient(zq - z)
    return (z_q_ste, loss, nearest)


# speedup vs baseline: 1.3008x; 1.3008x over previous
import jax
import jax.numpy as jnp
from jax.experimental import pallas as pl
from jax.experimental.pallas import tpu as pltpu

N = 16384
K = 1024
D = 64
BETA = 0.25
BLK = 1024


def _tc_body(z_ref, et_ref, z2_ref, e2_ref, e_ref,
             nearest_ref, zq_ref, loss_ref):
    i = pl.program_id(0)
    zb = z_ref[...]
    m = jax.lax.dot_general(
        zb, et_ref[...], (((1,), (0,)), ((), ())),
        preferred_element_type=jnp.float32)
    t1 = z2_ref[...] + e2_ref[...]
    d2 = t1 - 2.0 * m
    d = jnp.sqrt(jnp.maximum(d2, 0.0))
    dmin_keep = jnp.min(d, axis=1, keepdims=True)
    kiota = jax.lax.broadcasted_iota(jnp.int32, (BLK, K), 1)
    cand = jnp.where(d == dmin_keep, kiota, K)
    nearest = jnp.min(cand, axis=1)
    nearest_ref[0, ...] = nearest.reshape(1, BLK)
    onehot = (jax.lax.broadcasted_iota(jnp.int32, (BLK, K), 1)
              == nearest[:, None]).astype(jnp.float32)
    zq_ref[...] = jax.lax.dot_general(
        onehot, e_ref[...], (((1,), (0,)), ((), ())),
        preferred_element_type=jnp.float32)
    d2min = jnp.min(jnp.maximum(d2, 0.0), axis=1)

    @pl.when(i == 0)
    def _():
        loss_ref[0, 0] = 0.0

    loss_ref[0, 0] += jnp.sum(d2min)


def kernel(z, embeddings):
    z2 = jnp.sum(z * z, axis=1, keepdims=True)
    e2 = jnp.sum(embeddings * embeddings, axis=1)[None, :]
    et = embeddings.T
    grid = N // BLK
    nearest3, zq, loss_sum = pl.pallas_call(
        _tc_body,
        grid=(grid,),
        in_specs=[
            pl.BlockSpec((BLK, D), lambda i: (i, 0)),
            pl.BlockSpec((D, K), lambda i: (0, 0)),
            pl.BlockSpec((BLK, 1), lambda i: (i, 0)),
            pl.BlockSpec((1, K), lambda i: (0, 0)),
            pl.BlockSpec((K, D), lambda i: (0, 0)),
        ],
        out_specs=[
            pl.BlockSpec((1, 1, BLK), lambda i: (i, 0, 0)),
            pl.BlockSpec((BLK, D), lambda i: (i, 0)),
            pl.BlockSpec(memory_space=pltpu.SMEM),
        ],
        out_shape=[
            jax.ShapeDtypeStruct((grid, 1, BLK), jnp.int32),
            jax.ShapeDtypeStruct((N, D), jnp.float32),
            jax.ShapeDtypeStruct((1, 1), jnp.float32),
        ],
    )(z, et, z2, e2, embeddings)
    nearest = nearest3.reshape(N)
    loss = loss_sum[0, 0] * ((1.0 + BETA) / (N * D))
    z_q_ste = z + jax.lax.stop_gradient(zq - z)
    return (z_q_ste, loss, nearest)


# speedup vs baseline: 1.3126x; 1.0091x over previous
import jax
import jax.numpy as jnp
from jax.experimental import pallas as pl
from jax.experimental.pallas import tpu as pltpu

N = 16384
K = 1024
D = 64
BETA = 0.25
BLK = 2048


def _tc_body(z_ref, et_ref, z2_ref, e2_ref, e_ref,
             nearest_ref, zq_ref, loss_ref):
    i = pl.program_id(0)
    zb = z_ref[...]
    m = jax.lax.dot_general(
        zb, et_ref[...], (((1,), (0,)), ((), ())),
        preferred_element_type=jnp.float32)
    t1 = z2_ref[...] + e2_ref[...]
    d2 = t1 - 2.0 * m
    d = jnp.sqrt(jnp.maximum(d2, 0.0))
    dmin_keep = jnp.min(d, axis=1, keepdims=True)
    kiota = jax.lax.broadcasted_iota(jnp.int32, (BLK, K), 1)
    cand = jnp.where(d == dmin_keep, kiota, K)
    nearest = jnp.min(cand, axis=1)
    nearest_ref[0, ...] = nearest.reshape(1, BLK)
    onehot = (jax.lax.broadcasted_iota(jnp.int32, (BLK, K), 1)
              == nearest[:, None]).astype(jnp.float32)
    zq_ref[...] = jax.lax.dot_general(
        onehot, e_ref[...], (((1,), (0,)), ((), ())),
        preferred_element_type=jnp.float32)
    dmin = dmin_keep[:, 0]
    d2min = dmin * dmin

    @pl.when(i == 0)
    def _():
        loss_ref[0, 0] = 0.0

    loss_ref[0, 0] += jnp.sum(d2min)


def kernel(z, embeddings):
    z2 = jnp.sum(z * z, axis=1, keepdims=True)
    e2 = jnp.sum(embeddings * embeddings, axis=1)[None, :]
    et = embeddings.T
    grid = N // BLK
    nearest3, zq, loss_sum = pl.pallas_call(
        _tc_body,
        grid=(grid,),
        in_specs=[
            pl.BlockSpec((BLK, D), lambda i: (i, 0)),
            pl.BlockSpec((D, K), lambda i: (0, 0)),
            pl.BlockSpec((BLK, 1), lambda i: (i, 0)),
            pl.BlockSpec((1, K), lambda i: (0, 0)),
            pl.BlockSpec((K, D), lambda i: (0, 0)),
        ],
        out_specs=[
            pl.BlockSpec((1, 1, BLK), lambda i: (i, 0, 0)),
            pl.BlockSpec((BLK, D), lambda i: (i, 0)),
            pl.BlockSpec(memory_space=pltpu.SMEM),
        ],
        out_shape=[
            jax.ShapeDtypeStruct((grid, 1, BLK), jnp.int32),
            jax.ShapeDtypeStruct((N, D), jnp.float32),
            jax.ShapeDtypeStruct((1, 1), jnp.float32),
        ],
    )(z, et, z2, e2, embeddings)
    nearest = nearest3.reshape(N)
    loss = loss_sum[0, 0] * ((1.0 + BETA) / (N * D))
    z_q_ste = z + jax.lax.stop_gradient(zq - z)
    return (z_q_ste, loss, nearest)


# speedup vs baseline: 1.5296x; 1.1653x over previous
import jax
import jax.numpy as jnp
from jax.experimental import pallas as pl
from jax.experimental.pallas import tpu as pltpu

N = 16384
K = 1024
D = 64
BETA = 0.25
BLK = 2048


def _tc_body(z_ref, et_ref, z2_ref, e2_ref, e_ref,
             nearest_ref, zq_ref, loss_ref):
    i = pl.program_id(0)
    zb = z_ref[...]
    m = jax.lax.dot_general(
        zb, et_ref[...], (((1,), (0,)), ((), ())),
        preferred_element_type=jnp.float32)
    t1 = z2_ref[...] + e2_ref[...]
    d2 = t1 - 2.0 * m
    d = jnp.sqrt(jnp.maximum(d2, 0.0))
    dmin_keep = jnp.min(d, axis=1, keepdims=True)
    kiota_f = jax.lax.broadcasted_iota(jnp.int32, (BLK, K), 1).astype(jnp.float32)
    cand = jnp.where(d == dmin_keep, kiota_f, float(K))
    nearest_f = jnp.min(cand, axis=1)
    nearest_ref[0, ...] = nearest_f.astype(jnp.int32).reshape(1, BLK)
    onehot = (kiota_f == nearest_f[:, None]).astype(jnp.float32)
    zq = jax.lax.dot_general(
        onehot, e_ref[...], (((1,), (0,)), ((), ())),
        preferred_element_type=jnp.float32)
    zq_ref[...] = zb + (zq - zb)
    dmin = dmin_keep[:, 0]
    d2min = dmin * dmin

    @pl.when(i == 0)
    def _():
        loss_ref[0, 0] = 0.0

    loss_ref[0, 0] += jnp.sum(d2min)


def kernel(z, embeddings):
    z2 = jnp.sum(z * z, axis=1, keepdims=True)
    e2 = jnp.sum(embeddings * embeddings, axis=1)[None, :]
    et = embeddings.T
    grid = N // BLK
    nearest3, zq, loss_sum = pl.pallas_call(
        _tc_body,
        grid=(grid,),
        in_specs=[
            pl.BlockSpec((BLK, D), lambda i: (i, 0)),
            pl.BlockSpec((D, K), lambda i: (0, 0)),
            pl.BlockSpec((BLK, 1), lambda i: (i, 0)),
            pl.BlockSpec((1, K), lambda i: (0, 0)),
            pl.BlockSpec((K, D), lambda i: (0, 0)),
        ],
        out_specs=[
            pl.BlockSpec((1, 1, BLK), lambda i: (i, 0, 0)),
            pl.BlockSpec((BLK, D), lambda i: (i, 0)),
            pl.BlockSpec(memory_space=pltpu.SMEM),
        ],
        out_shape=[
            jax.ShapeDtypeStruct((grid, 1, BLK), jnp.int32),
            jax.ShapeDtypeStruct((N, D), jnp.float32),
            jax.ShapeDtypeStruct((1, 1), jnp.float32),
        ],
    )(z, et, z2, e2, embeddings)
    nearest = nearest3.reshape(N)
    loss = loss_sum[0, 0] * ((1.0 + BETA) / (N * D))
    return (zq, loss, nearest)
